# Initial kernel scaffold; baseline (speedup 1.0000x reference)
import jax
import jax.numpy as jnp
from jax.experimental import pallas as pl


def kernel(inputs, table):
    raise NotImplementedError("write your pallas kernel here")


# speedup vs baseline: 3.5926x; 3.5926x over previous
import functools
import math

import jax
import jax.numpy as jnp
from jax import lax
from jax.experimental import pallas as pl
from jax.experimental.pallas import tpu as pltpu
from jax.experimental.pallas import tpu_sc as plsc

_D = 128
_SCALE = math.sqrt(float(_D))


def _pe_table(pos, d_embed):
    i = jnp.arange(d_embed // 2, dtype=jnp.float32)
    angle = (jnp.arange(pos, dtype=jnp.float32)[:, None]
             / jnp.power(10000.0, 2.0 * i / d_embed)[None, :])
    enc = jnp.concatenate([jnp.sin(angle)[:, :, None], jnp.cos(angle)[:, :, None]],
                          axis=-1)
    return jnp.reshape(enc, (-1, d_embed))


def _make_sc_kernel(n_rows, seq, d, n_workers):
    rows_per_w = n_rows // n_workers
    chunk = seq
    n_chunks = rows_per_w // chunk
    mesh = plsc.VectorSubcoreMesh(core_axis_name="c", subcore_axis_name="s")

    @functools.partial(
        pl.kernel,
        out_type=jax.ShapeDtypeStruct((n_rows, d), jnp.float32),
        mesh=mesh,
        scratch_types=[
            pltpu.VMEM((chunk,), jnp.int32),
            pltpu.VMEM((chunk, d), jnp.float32),
            pltpu.VMEM((seq, d), jnp.float32),
            pltpu.SemaphoreType.DMA,
        ],
    )
    def sc_kernel(idx_hbm, table_hbm, pe_hbm, out_hbm, idx_v, rows_v, pe_v, sem):
        nc = lax.axis_size("c")
        wid = lax.axis_index("s") * nc + lax.axis_index("c")
        base = wid * rows_per_w
        pltpu.sync_copy(pe_hbm, pe_v)

        def chunk_body(k, carry):
            row0 = base + k * chunk
            pltpu.sync_copy(idx_hbm.at[pl.ds(row0, chunk)], idx_v)
            pltpu.async_copy(table_hbm.at[idx_v.at[pl.ds(0, 128)]],
                             rows_v.at[pl.ds(0, 128)], sem).wait()
            pltpu.async_copy(table_hbm.at[idx_v.at[pl.ds(128, chunk - 128)]],
                             rows_v.at[pl.ds(128, chunk - 128)], sem).wait()

            def row_body(r, c2):
                for c in range(d // 16):
                    sl = pl.ds(c * 16, 16)
                    rows_v[r, sl] = rows_v[r, sl] * _SCALE + pe_v[r, sl]
                return c2

            lax.fori_loop(0, chunk, row_body, 0, unroll=False)
            pltpu.sync_copy(rows_v, out_hbm.at[pl.ds(row0, chunk)])
            return carry

        lax.fori_loop(0, n_chunks, chunk_body, 0, unroll=False)

    return sc_kernel


def kernel(inputs, table):
    b, s = inputs.shape
    v, d = table.shape
    n_rows = b * s
    idx = inputs.reshape(n_rows).astype(jnp.int32)
    pe = _pe_table(s, d)
    info = plsc.get_sparse_core_info()
    n_workers = info.num_cores * info.num_subcores
    out = _make_sc_kernel(n_rows, s, d, n_workers)(idx, table, pe)
    return out.reshape(b, s, d)
